# baseline (speedup 1.0000x reference)
import functools
import math

import jax
import jax.numpy as jnp
from jax import lax
from jax.experimental import pallas as pl
from jax.experimental.pallas import tpu as pltpu
from jax.experimental.pallas import tpu_sc as plsc

D = 512
L_SEQ = 512
SCALE = math.sqrt(float(D))

try:
    _info = plsc.get_sparse_core_info()
    NC, NS, LANES = _info.num_cores, _info.num_subcores, _info.num_lanes
except Exception:
    NC, NS, LANES = 2, 16, 16
NW = NC * NS


def _make_lookup(total_rows: int, chunk_rows: int):
    rows_w = total_rows // NW
    nchunk = rows_w // chunk_rows
    half = total_rows // 2
    vpc = chunk_rows * D // LANES

    mesh = plsc.VectorSubcoreMesh(core_axis_name="c", subcore_axis_name="s")

    @functools.partial(
        pl.kernel,
        mesh=mesh,
        out_type=jax.ShapeDtypeStruct((total_rows, D), jnp.float32),
        scratch_types=[
            pltpu.VMEM((nchunk, chunk_rows), jnp.int32),
            pltpu.VMEM((chunk_rows, D), jnp.float32),
            pltpu.VMEM((chunk_rows, D), jnp.float32),
            pltpu.SemaphoreType.DMA,
        ],
    )
    def lookup(idx_hbm, table_hbm, pos_hbm, out_hbm, idx_v, emb_v, pos_v, sem):
        wid = lax.axis_index("s") * NC + lax.axis_index("c")
        base = wid * rows_w
        s_stack = base // half
        pltpu.sync_copy(idx_hbm.at[pl.ds(wid * nchunk, nchunk)], idx_v)

        def chunk_body(c, carry):
            row0 = base + c * chunk_rows
            pos0 = s_stack * L_SEQ + (c * chunk_rows) % L_SEQ
            pltpu.sync_copy(pos_hbm.at[pl.ds(pos0, chunk_rows)], pos_v)
            pltpu.async_copy(table_hbm.at[idx_v.at[c]], emb_v, sem).wait()

            def vec_body(i, carry2):
                r = i // (D // LANES)
                col = (i % (D // LANES)) * LANES
                e = emb_v[r, pl.ds(col, LANES)]
                p = pos_v[r, pl.ds(col, LANES)]
                emb_v[r, pl.ds(col, LANES)] = e * SCALE + p
                return carry2

            lax.fori_loop(0, vpc, vec_body, 0)
            pltpu.sync_copy(emb_v, out_hbm.at[pl.ds(row0, chunk_rows)])
            return carry

        lax.fori_loop(0, nchunk, chunk_body, 0)

    return lookup


def kernel(src, tgt, emb_table, pos_src_table, pos_tgt_table):
    B, L = src.shape
    _, LP = tgt.shape
    total_rows = B * L + B * LP
    chunk_rows = 64
    idx_all = jnp.concatenate([src.reshape(-1), tgt.reshape(-1)])
    idx_2d = idx_all.reshape(total_rows // chunk_rows, chunk_rows)
    pos_cat = jnp.concatenate([pos_src_table, pos_tgt_table], axis=0)
    flat = _make_lookup(total_rows, chunk_rows)(idx_2d, emb_table, pos_cat)
    return flat.reshape(2, B, L, D)


# speedup vs baseline: 3.6526x; 3.6526x over previous
import functools
import math

import jax
import jax.numpy as jnp
from jax import lax
from jax.experimental import pallas as pl
from jax.experimental.pallas import tpu as pltpu
from jax.experimental.pallas import tpu_sc as plsc

D = 512
L_SEQ = 512
SCALE = math.sqrt(float(D))

try:
    _info = plsc.get_sparse_core_info()
    NC, NS, LANES = _info.num_cores, _info.num_subcores, _info.num_lanes
except Exception:
    NC, NS, LANES = 2, 16, 16
NW = NC * NS


def _make_lookup(total_rows: int, chunk_rows: int):
    rows_w = total_rows // NW
    nchunk = rows_w // chunk_rows
    half = total_rows // 2
    npairs = nchunk // 2

    mesh = plsc.VectorSubcoreMesh(core_axis_name="c", subcore_axis_name="s")

    @functools.partial(
        pl.kernel,
        mesh=mesh,
        out_type=jax.ShapeDtypeStruct((total_rows, D), jnp.float32),
        scratch_types=[
            pltpu.VMEM((nchunk, chunk_rows), jnp.int32),
            pltpu.VMEM((2, chunk_rows, D), jnp.float32),
            pltpu.VMEM((2, chunk_rows, D), jnp.float32),
            pltpu.SemaphoreType.DMA,
            pltpu.SemaphoreType.DMA,
            pltpu.SemaphoreType.DMA,
            pltpu.SemaphoreType.DMA,
            pltpu.SemaphoreType.DMA,
            pltpu.SemaphoreType.DMA,
        ],
    )
    def lookup(idx_hbm, table_hbm, pos_hbm, out_hbm,
               idx_v, ebuf, pbuf, g0, g1, p0, p1, o0, o1):
        wid = lax.axis_index("s") * NC + lax.axis_index("c")
        base = wid * rows_w
        s_stack = base // half
        gsem = (g0, g1)
        psem = (p0, p1)
        osem = (o0, o1)

        pltpu.sync_copy(idx_hbm.at[pl.ds(wid * nchunk, nchunk)], idx_v)

        def issue_in(c, b):
            pltpu.async_copy(table_hbm.at[idx_v.at[c]], ebuf.at[b], gsem[b])
            pos0 = s_stack * L_SEQ + (c * chunk_rows) % L_SEQ
            pltpu.async_copy(pos_hbm.at[pl.ds(pos0, chunk_rows)],
                             pbuf.at[b], psem[b])

        def wait_in(c, b):
            pltpu.make_async_copy(table_hbm.at[idx_v.at[c]],
                                  ebuf.at[b], gsem[b]).wait()
            pltpu.make_async_copy(pos_hbm.at[pl.ds(s_stack * L_SEQ, chunk_rows)],
                                  pbuf.at[b], psem[b]).wait()

        def issue_out(c, b):
            pltpu.async_copy(ebuf.at[b], out_hbm.at[pl.ds(base + c * chunk_rows,
                                                          chunk_rows)], osem[b])

        def wait_out(b):
            pltpu.make_async_copy(ebuf.at[b],
                                  out_hbm.at[pl.ds(base, chunk_rows)],
                                  osem[b]).wait()

        def combine(b):
            eb = ebuf.at[b]
            pb = pbuf.at[b]

            def row_body(r, carry):
                for j in range(D // LANES):
                    sl = pl.ds(j * LANES, LANES)
                    eb[r, sl] = eb[r, sl] * SCALE + pb[r, sl]
                return carry

            lax.fori_loop(0, chunk_rows, row_body, 0)

        issue_in(0, 0)

        def pair_body(i, carry):
            for b in range(2):
                c = 2 * i + b
                nb = 1 - b

                @pl.when(c + 1 < nchunk)
                def _():
                    @pl.when(c >= 1)
                    def _():
                        wait_out(nb)
                    issue_in(c + 1, nb)

                wait_in(c, b)
                combine(b)
                issue_out(c, b)
            return carry

        lax.fori_loop(0, npairs, pair_body, 0)
        wait_out(0)
        wait_out(1)

    return lookup


def kernel(src, tgt, emb_table, pos_src_table, pos_tgt_table):
    B, L = src.shape
    _, LP = tgt.shape
    total_rows = B * L + B * LP
    chunk_rows = 32
    idx_all = jnp.concatenate([src.reshape(-1), tgt.reshape(-1)])
    idx_2d = idx_all.reshape(total_rows // chunk_rows, chunk_rows)
    pos_cat = jnp.concatenate([pos_src_table, pos_tgt_table], axis=0)
    flat = _make_lookup(total_rows, chunk_rows)(idx_2d, emb_table, pos_cat)
    return flat.reshape(2, B, L, D)
